# baseline (speedup 1.0000x reference)
import functools

import jax
import jax.numpy as jnp
from jax import lax
from jax.experimental import pallas as pl
from jax.experimental.pallas import tpu as pltpu
from jax.experimental.pallas import tpu_sc as plsc

N = 10000
E = 160000
D = 256
D_OUT = 64
H = 128
NC = 2
NS = 16
L = 16
CHUNK = 128
CH_SEG = 80
PH_SEG = 2
CH_PH = CH_SEG // PH_SEG
CH_DEG = 40
EP = NS * CH_SEG * CHUNK
ACC = 10112
RPT = ACC // NS

_mesh = plsc.VectorSubcoreMesh(
    core_axis_name="c", subcore_axis_name="s", num_cores=NC, num_subcores=NS)

_ZCHUNKS = (128, 128, 128, 128, 120)


@functools.partial(
    pl.kernel,
    out_type=jax.ShapeDtypeStruct((NC, ACC, H), jnp.float32),
    mesh=_mesh,
    scratch_types=[
        pltpu.VMEM_SHARED((ACC, H), jnp.float32),
        pltpu.VMEM((CH_DEG, CHUNK), jnp.int32),
        pltpu.VMEM((CHUNK, H), jnp.float32),
        pltpu.VMEM((CHUNK, H), jnp.float32),
    ],
)
def _sc_degree(dsts_hbm, out_hbm, acc, didx, obuf, zbuf):
    c = lax.axis_index("c")
    s = lax.axis_index("s")
    w = c * NS + s

    def fill(r, _):
        for j in range(H // L):
            obuf[r, pl.ds(j * L, L)] = jnp.ones((L,), jnp.float32)
            zbuf[r, pl.ds(j * L, L)] = jnp.zeros((L,), jnp.float32)
        return 0
    lax.fori_loop(0, CHUNK, fill, 0)

    base = s * RPT
    off = 0
    for ln_ in _ZCHUNKS:
        pltpu.sync_copy(zbuf.at[pl.ds(0, ln_)], acc.at[pl.ds(base + off, ln_)])
        off += ln_
    pltpu.sync_copy(dsts_hbm.at[w], didx)
    plsc.subcore_barrier()

    def body(j, _):
        pltpu.sync_copy(obuf, acc.at[didx.at[j]], add=True)
        return 0
    lax.fori_loop(0, CH_DEG, body, 0)
    plsc.subcore_barrier()
    pltpu.sync_copy(acc.at[pl.ds(base, RPT)], out_hbm.at[c, pl.ds(base, RPT)])


@functools.partial(
    pl.kernel,
    out_type=jax.ShapeDtypeStruct((NC, ACC, H), jnp.float32),
    mesh=_mesh,
    scratch_types=[
        pltpu.VMEM_SHARED((ACC, H), jnp.float32),
        pltpu.VMEM((CH_PH, CHUNK), jnp.int32),
        pltpu.VMEM((CH_PH, CHUNK), jnp.int32),
        pltpu.VMEM((CHUNK, H), jnp.float32),
        pltpu.VMEM((CHUNK, H), jnp.float32),
        pltpu.SemaphoreType.DMA,
        pltpu.SemaphoreType.DMA,
        pltpu.SemaphoreType.DMA,
        pltpu.SemaphoreType.DMA,
    ],
)
def _sc_segment(g_hbm, srcs_hbm, dsts_hbm, out_hbm,
                acc, sidx, didx, rows0, rows1, sem0, sem1, ssem0, ssem1):
    c = lax.axis_index("c")
    s = lax.axis_index("s")

    def zrow(r, _):
        for j in range(H // L):
            rows0[r, pl.ds(j * L, L)] = jnp.zeros((L,), jnp.float32)
        return 0
    lax.fori_loop(0, CHUNK, zrow, 0)

    base = s * RPT
    off = 0
    for ln_ in _ZCHUNKS:
        pltpu.sync_copy(rows0.at[pl.ds(0, ln_)], acc.at[pl.ds(base + off, ln_)])
        off += ln_
    plsc.subcore_barrier()

    def body(jj, _):
        j0 = jj * 2
        j1 = j0 + 1
        cp0 = pltpu.async_copy(g_hbm.at[sidx.at[j0]], rows0, sem0)
        cp1 = pltpu.async_copy(g_hbm.at[sidx.at[j1]], rows1, sem1)
        cp0.wait()
        sc0 = pltpu.async_copy(rows0, acc.at[didx.at[j0]], ssem0, add=True)
        cp1.wait()
        sc1 = pltpu.async_copy(rows1, acc.at[didx.at[j1]], ssem1, add=True)
        sc0.wait()
        sc1.wait()
        return 0

    for ph in range(PH_SEG):
        pltpu.sync_copy(srcs_hbm.at[c, s, pl.ds(ph * CH_PH, CH_PH)], sidx)
        pltpu.sync_copy(dsts_hbm.at[s, pl.ds(ph * CH_PH, CH_PH)], didx)
        lax.fori_loop(0, CH_PH // 2, body, 0)
    plsc.subcore_barrier()
    pltpu.sync_copy(acc.at[pl.ds(base, RPT)], out_hbm.at[c, pl.ds(base, RPT)])


R_TC = 1000
GRID = N // R_TC
_f32 = jnp.float32


def _tc_matmul0_body(x_ref, w_ref, hw_ref):
    hw_ref[...] = jnp.dot(x_ref[...], w_ref[...], preferred_element_type=_f32)


_tc_matmul0 = pl.pallas_call(
    _tc_matmul0_body,
    grid=(GRID,),
    in_specs=[
        pl.BlockSpec((R_TC, D), lambda i: (i, 0)),
        pl.BlockSpec((D, D), lambda i: (0, 0)),
    ],
    out_specs=pl.BlockSpec((R_TC, D), lambda i: (i, 0)),
    out_shape=jax.ShapeDtypeStruct((N, D), _f32),
)


def _tc_first_body(hw_ref, d0_ref, d1_ref, g_ref, dis_ref):
    dis = lax.rsqrt(d0_ref[...] + d1_ref[...] + 1.0)
    g = hw_ref[...] * dis
    g_ref[0] = g[:, :H]
    g_ref[1] = g[:, H:]
    dis_ref[...] = dis


_tc_first = pl.pallas_call(
    _tc_first_body,
    grid=(GRID,),
    in_specs=[
        pl.BlockSpec((R_TC, D), lambda i: (i, 0)),
        pl.BlockSpec((R_TC, 1), lambda i: (i, 0)),
        pl.BlockSpec((R_TC, 1), lambda i: (i, 0)),
    ],
    out_specs=[
        pl.BlockSpec((NC, R_TC, H), lambda i: (0, i, 0)),
        pl.BlockSpec((R_TC, 1), lambda i: (i, 0)),
    ],
    out_shape=[
        jax.ShapeDtypeStruct((NC, N, H), _f32),
        jax.ShapeDtypeStruct((N, 1), _f32),
    ],
)


def _tc_mid_body(s_ref, g_ref, dis_ref, b_ref, lg_ref, lb_ref, w_ref, go_ref):
    u = jnp.concatenate([s_ref[0] + g_ref[0], s_ref[1] + g_ref[1]], axis=1)
    dis = dis_ref[...]
    h = jnp.maximum(u * dis + b_ref[...], 0.0)
    mu = jnp.mean(h, axis=1, keepdims=True)
    xc = h - mu
    var = jnp.mean(xc * xc, axis=1, keepdims=True)
    hn = xc * lax.rsqrt(var + 1e-5) * lg_ref[...] + lb_ref[...]
    gn = jnp.dot(hn, w_ref[...], preferred_element_type=_f32) * dis
    go_ref[0] = gn[:, :H]
    go_ref[1] = gn[:, H:]


_tc_mid = pl.pallas_call(
    _tc_mid_body,
    grid=(GRID,),
    in_specs=[
        pl.BlockSpec((NC, R_TC, H), lambda i: (0, i, 0)),
        pl.BlockSpec((NC, R_TC, H), lambda i: (0, i, 0)),
        pl.BlockSpec((R_TC, 1), lambda i: (i, 0)),
        pl.BlockSpec((1, D), lambda i: (0, 0)),
        pl.BlockSpec((1, D), lambda i: (0, 0)),
        pl.BlockSpec((1, D), lambda i: (0, 0)),
        pl.BlockSpec((D, D), lambda i: (0, 0)),
    ],
    out_specs=pl.BlockSpec((NC, R_TC, H), lambda i: (0, i, 0)),
    out_shape=jax.ShapeDtypeStruct((NC, N, H), _f32),
)


def _tc_final_body(s_ref, g_ref, dis_ref, b_ref, w1_ref, b1_ref, w2_ref,
                   b2_ref, emb_ref, lp_ref):
    u = jnp.concatenate([s_ref[0] + g_ref[0], s_ref[1] + g_ref[1]], axis=1)
    emb = u * dis_ref[...] + b_ref[...]
    h = jnp.maximum(emb, 0.0)
    t = jnp.dot(h, w1_ref[...], preferred_element_type=_f32) + b1_ref[...]
    o = jnp.dot(t, w2_ref[...], preferred_element_type=_f32) + b2_ref[...]
    m = jnp.max(o, axis=1, keepdims=True)
    e = o - m
    lse = jnp.log(jnp.sum(jnp.exp(e), axis=1, keepdims=True))
    emb_ref[...] = emb
    lp_ref[...] = e - lse


_tc_final = pl.pallas_call(
    _tc_final_body,
    grid=(GRID,),
    in_specs=[
        pl.BlockSpec((NC, R_TC, H), lambda i: (0, i, 0)),
        pl.BlockSpec((NC, R_TC, H), lambda i: (0, i, 0)),
        pl.BlockSpec((R_TC, 1), lambda i: (i, 0)),
        pl.BlockSpec((1, D), lambda i: (0, 0)),
        pl.BlockSpec((D, D), lambda i: (0, 0)),
        pl.BlockSpec((1, D), lambda i: (0, 0)),
        pl.BlockSpec((D, D_OUT), lambda i: (0, 0)),
        pl.BlockSpec((1, D_OUT), lambda i: (0, 0)),
    ],
    out_specs=[
        pl.BlockSpec((R_TC, D), lambda i: (i, 0)),
        pl.BlockSpec((R_TC, D_OUT), lambda i: (i, 0)),
    ],
    out_shape=[
        jax.ShapeDtypeStruct((N, D), _f32),
        jax.ShapeDtypeStruct((N, D_OUT), _f32),
    ],
)


def kernel(x, edge_index, batch, W1, b1, W2, b2, W3, b3,
           ln1_g, ln1_b, ln2_g, ln2_b, pW1, pb1, pW2, pb2):
    src = edge_index[0].astype(jnp.int32)
    dst = edge_index[1].astype(jnp.int32)
    pad = EP - E
    srcp = jnp.concatenate([src, jnp.zeros((pad,), jnp.int32)])
    dstp = jnp.concatenate([dst, jnp.full((pad,), N, jnp.int32)])
    srcs = jnp.stack([srcp, srcp + N]).reshape(NC, NS, CH_SEG, CHUNK)
    dsts16 = dstp.reshape(NS, CH_SEG, CHUNK)
    dsts32 = dstp.reshape(NC * NS, CH_DEG, CHUNK)

    degp = _sc_degree(dsts32)
    hw0 = _tc_matmul0(x, W1)
    d0 = degp[0, :N, 0:1]
    d1 = degp[1, :N, 0:1]

    g0, dis = _tc_first(hw0, d0, d1)
    S0 = _sc_segment(g0.reshape(NC * N, H), srcs, dsts16)
    g1 = _tc_mid(S0, g0, dis, b1.reshape(1, D), ln1_g.reshape(1, D),
                 ln1_b.reshape(1, D), W2)
    S1 = _sc_segment(g1.reshape(NC * N, H), srcs, dsts16)
    g2 = _tc_mid(S1, g1, dis, b2.reshape(1, D), ln2_g.reshape(1, D),
                 ln2_b.reshape(1, D), W3)
    S2 = _sc_segment(g2.reshape(NC * N, H), srcs, dsts16)
    emb, logp = _tc_final(S2, g2, dis, b3.reshape(1, D), pW1,
                          pb1.reshape(1, D), pW2, pb2.reshape(1, D_OUT))
    return emb, logp


# speedup vs baseline: 1.0219x; 1.0219x over previous
import functools

import jax
import jax.numpy as jnp
from jax import lax
from jax.experimental import pallas as pl
from jax.experimental.pallas import tpu as pltpu
from jax.experimental.pallas import tpu_sc as plsc

N = 10000
E = 160000
D = 256
D_OUT = 64
H = 128
NC = 2
NS = 16
L = 16
CHUNK = 128
CH_SEG = 80
PH_SEG = 2
CH_PH = CH_SEG // PH_SEG
CH_DEG = 40
EP = NS * CH_SEG * CHUNK
ACC = 10112
RPT = ACC // NS

_mesh = plsc.VectorSubcoreMesh(
    core_axis_name="c", subcore_axis_name="s", num_cores=NC, num_subcores=NS)

_ZCHUNKS = (128, 128, 128, 128, 120)


@functools.partial(
    pl.kernel,
    out_type=jax.ShapeDtypeStruct((NC, ACC, H), jnp.float32),
    mesh=_mesh,
    scratch_types=[
        pltpu.VMEM_SHARED((ACC, H), jnp.float32),
        pltpu.VMEM((CH_DEG, CHUNK), jnp.int32),
        pltpu.VMEM((CHUNK, H), jnp.float32),
        pltpu.VMEM((CHUNK, H), jnp.float32),
    ],
)
def _sc_degree(dsts_hbm, out_hbm, acc, didx, obuf, zbuf):
    c = lax.axis_index("c")
    s = lax.axis_index("s")
    w = c * NS + s

    def fill(r, _):
        for j in range(H // L):
            obuf[r, pl.ds(j * L, L)] = jnp.ones((L,), jnp.float32)
            zbuf[r, pl.ds(j * L, L)] = jnp.zeros((L,), jnp.float32)
        return 0
    lax.fori_loop(0, CHUNK, fill, 0)

    base = s * RPT
    off = 0
    for ln_ in _ZCHUNKS:
        pltpu.sync_copy(zbuf.at[pl.ds(0, ln_)], acc.at[pl.ds(base + off, ln_)])
        off += ln_
    pltpu.sync_copy(dsts_hbm.at[w], didx)
    plsc.subcore_barrier()

    def body(j, _):
        pltpu.sync_copy(obuf, acc.at[didx.at[j]], add=True)
        return 0
    lax.fori_loop(0, CH_DEG, body, 0)
    plsc.subcore_barrier()
    pltpu.sync_copy(acc.at[pl.ds(base, RPT)], out_hbm.at[c, pl.ds(base, RPT)])


@functools.partial(
    pl.kernel,
    out_type=jax.ShapeDtypeStruct((NC, ACC, H), jnp.float32),
    mesh=_mesh,
    scratch_types=[
        pltpu.VMEM_SHARED((ACC, H), jnp.float32),
        pltpu.VMEM((CH_PH, CHUNK), jnp.int32),
        pltpu.VMEM((CH_PH, CHUNK), jnp.int32),
        pltpu.VMEM((CHUNK, H), jnp.float32),
        pltpu.VMEM((CHUNK, H), jnp.float32),
        pltpu.SemaphoreType.DMA,
        pltpu.SemaphoreType.DMA,
        pltpu.SemaphoreType.DMA,
        pltpu.SemaphoreType.DMA,
    ],
)
def _sc_segment(g_hbm, srcs_hbm, dsts_hbm, out_hbm,
                acc, sidx, didx, rows0, rows1, sem0, sem1, ssem0, ssem1):
    c = lax.axis_index("c")
    s = lax.axis_index("s")

    def zrow(r, _):
        for j in range(H // L):
            rows0[r, pl.ds(j * L, L)] = jnp.zeros((L,), jnp.float32)
        return 0
    lax.fori_loop(0, CHUNK, zrow, 0)

    base = s * RPT
    off = 0
    for ln_ in _ZCHUNKS:
        pltpu.sync_copy(rows0.at[pl.ds(0, ln_)], acc.at[pl.ds(base + off, ln_)])
        off += ln_
    plsc.subcore_barrier()

    def body(jj, _):
        j0 = jj * 2
        j1 = j0 + 1
        cp0 = pltpu.async_copy(g_hbm.at[sidx.at[j0]], rows0, sem0)
        cp1 = pltpu.async_copy(g_hbm.at[sidx.at[j1]], rows1, sem1)
        cp0.wait()
        sc0 = pltpu.async_copy(rows0, acc.at[didx.at[j0]], ssem0, add=True)
        cp1.wait()
        sc1 = pltpu.async_copy(rows1, acc.at[didx.at[j1]], ssem1, add=True)
        sc0.wait()
        sc1.wait()
        return 0

    for ph in range(PH_SEG):
        pltpu.sync_copy(srcs_hbm.at[c, s, pl.ds(ph * CH_PH, CH_PH)], sidx)
        pltpu.sync_copy(dsts_hbm.at[s, pl.ds(ph * CH_PH, CH_PH)], didx)
        lax.fori_loop(0, CH_PH // 2, body, 0)
    plsc.subcore_barrier()
    pltpu.sync_copy(acc.at[pl.ds(base, RPT)], out_hbm.at[c, pl.ds(base, RPT)])


R_TC = 2000
GRID = N // R_TC
_f32 = jnp.float32


def _tc_first_body(x_ref, w_ref, d0_ref, d1_ref, g_ref, dis_ref):
    dis = lax.rsqrt(d0_ref[...] + d1_ref[...] + 1.0)
    g = jnp.dot(x_ref[...], w_ref[...], preferred_element_type=_f32) * dis
    g_ref[0] = g[:, :H]
    g_ref[1] = g[:, H:]
    dis_ref[...] = dis


_tc_first = pl.pallas_call(
    _tc_first_body,
    grid=(GRID,),
    in_specs=[
        pl.BlockSpec((R_TC, D), lambda i: (i, 0)),
        pl.BlockSpec((D, D), lambda i: (0, 0)),
        pl.BlockSpec((R_TC, 1), lambda i: (i, 0)),
        pl.BlockSpec((R_TC, 1), lambda i: (i, 0)),
    ],
    out_specs=[
        pl.BlockSpec((NC, R_TC, H), lambda i: (0, i, 0)),
        pl.BlockSpec((R_TC, 1), lambda i: (i, 0)),
    ],
    out_shape=[
        jax.ShapeDtypeStruct((NC, N, H), _f32),
        jax.ShapeDtypeStruct((N, 1), _f32),
    ],
)


def _tc_mid_body(s_ref, g_ref, dis_ref, b_ref, lg_ref, lb_ref, w_ref, go_ref):
    u = jnp.concatenate([s_ref[0] + g_ref[0], s_ref[1] + g_ref[1]], axis=1)
    dis = dis_ref[...]
    h = jnp.maximum(u * dis + b_ref[...], 0.0)
    mu = jnp.mean(h, axis=1, keepdims=True)
    xc = h - mu
    var = jnp.mean(xc * xc, axis=1, keepdims=True)
    hn = xc * lax.rsqrt(var + 1e-5) * lg_ref[...] + lb_ref[...]
    gn = jnp.dot(hn, w_ref[...], preferred_element_type=_f32) * dis
    go_ref[0] = gn[:, :H]
    go_ref[1] = gn[:, H:]


_tc_mid = pl.pallas_call(
    _tc_mid_body,
    grid=(GRID,),
    in_specs=[
        pl.BlockSpec((NC, R_TC, H), lambda i: (0, i, 0)),
        pl.BlockSpec((NC, R_TC, H), lambda i: (0, i, 0)),
        pl.BlockSpec((R_TC, 1), lambda i: (i, 0)),
        pl.BlockSpec((1, D), lambda i: (0, 0)),
        pl.BlockSpec((1, D), lambda i: (0, 0)),
        pl.BlockSpec((1, D), lambda i: (0, 0)),
        pl.BlockSpec((D, D), lambda i: (0, 0)),
    ],
    out_specs=pl.BlockSpec((NC, R_TC, H), lambda i: (0, i, 0)),
    out_shape=jax.ShapeDtypeStruct((NC, N, H), _f32),
)


def _tc_final_body(s_ref, g_ref, dis_ref, b_ref, w1_ref, b1_ref, w2_ref,
                   b2_ref, emb_ref, lp_ref):
    u = jnp.concatenate([s_ref[0] + g_ref[0], s_ref[1] + g_ref[1]], axis=1)
    emb = u * dis_ref[...] + b_ref[...]
    h = jnp.maximum(emb, 0.0)
    t = jnp.dot(h, w1_ref[...], preferred_element_type=_f32) + b1_ref[...]
    o = jnp.dot(t, w2_ref[...], preferred_element_type=_f32) + b2_ref[...]
    m = jnp.max(o, axis=1, keepdims=True)
    e = o - m
    lse = jnp.log(jnp.sum(jnp.exp(e), axis=1, keepdims=True))
    emb_ref[...] = emb
    lp_ref[...] = e - lse


_tc_final = pl.pallas_call(
    _tc_final_body,
    grid=(GRID,),
    in_specs=[
        pl.BlockSpec((NC, R_TC, H), lambda i: (0, i, 0)),
        pl.BlockSpec((NC, R_TC, H), lambda i: (0, i, 0)),
        pl.BlockSpec((R_TC, 1), lambda i: (i, 0)),
        pl.BlockSpec((1, D), lambda i: (0, 0)),
        pl.BlockSpec((D, D), lambda i: (0, 0)),
        pl.BlockSpec((1, D), lambda i: (0, 0)),
        pl.BlockSpec((D, D_OUT), lambda i: (0, 0)),
        pl.BlockSpec((1, D_OUT), lambda i: (0, 0)),
    ],
    out_specs=[
        pl.BlockSpec((R_TC, D), lambda i: (i, 0)),
        pl.BlockSpec((R_TC, D_OUT), lambda i: (i, 0)),
    ],
    out_shape=[
        jax.ShapeDtypeStruct((N, D), _f32),
        jax.ShapeDtypeStruct((N, D_OUT), _f32),
    ],
)


def kernel(x, edge_index, batch, W1, b1, W2, b2, W3, b3,
           ln1_g, ln1_b, ln2_g, ln2_b, pW1, pb1, pW2, pb2):
    src = edge_index[0].astype(jnp.int32)
    dst = edge_index[1].astype(jnp.int32)
    pad = EP - E
    srcp = jnp.concatenate([src, jnp.zeros((pad,), jnp.int32)])
    dstp = jnp.concatenate([dst, jnp.full((pad,), N, jnp.int32)])
    srcs = jnp.stack([srcp, srcp + N]).reshape(NC, NS, CH_SEG, CHUNK)
    dsts16 = dstp.reshape(NS, CH_SEG, CHUNK)
    dsts32 = dstp.reshape(NC * NS, CH_DEG, CHUNK)

    degp = _sc_degree(dsts32)
    d0 = degp[0, :N, 0:1]
    d1 = degp[1, :N, 0:1]

    g0, dis = _tc_first(x, W1, d0, d1)
    S0 = _sc_segment(g0.reshape(NC * N, H), srcs, dsts16)
    g1 = _tc_mid(S0, g0, dis, b1.reshape(1, D), ln1_g.reshape(1, D),
                 ln1_b.reshape(1, D), W2)
    S1 = _sc_segment(g1.reshape(NC * N, H), srcs, dsts16)
    g2 = _tc_mid(S1, g1, dis, b2.reshape(1, D), ln2_g.reshape(1, D),
                 ln2_b.reshape(1, D), W3)
    S2 = _sc_segment(g2.reshape(NC * N, H), srcs, dsts16)
    emb, logp = _tc_final(S2, g2, dis, b3.reshape(1, D), pW1,
                          pb1.reshape(1, D), pW2, pb2.reshape(1, D_OUT))
    return emb, logp
